# baseline (speedup 1.0000x reference)
import functools

import jax
import jax.numpy as jnp
from jax import lax
from jax.experimental import pallas as pl
from jax.experimental.pallas import tpu as pltpu
from jax.experimental.pallas import tpu_sc as plsc

_TOPK = 2
_NC, _NS, _L = 2, 16, 16


def _fused_body(x_ref, wr_ref, br_ref, we_ref, be_ref,
                final_ref, scores_out_ref,
                sel_ref, s_ref, const_ref):
    e = pl.program_id(0)
    E = pl.num_programs(0)
    B, D = x_ref.shape
    OUT = we_ref.shape[2]
    x = x_ref[...]

    @pl.when(e == 0)
    def _():
        scores = jax.lax.dot(x, wr_ref[...],
                             preferred_element_type=jnp.float32)
        scores = scores + br_ref[...]
        scores_t = jax.lax.dot_general(
            wr_ref[...], x, (((0,), (1,)), ((), ())),
            preferred_element_type=jnp.float32)
        scores_out_ref[...] = scores_t + br_ref[...].reshape(E, 1)
        iota_e = jax.lax.broadcasted_iota(jnp.int32, (B, E), 1)
        rank = jnp.zeros((B, E), jnp.float32)
        for ep in range(8):
            sp = scores[:, ep:ep + 1]
            gt = (sp > scores).astype(jnp.float32)
            eq = jnp.logical_and(sp == scores, ep < iota_e).astype(jnp.float32)
            rank = rank + gt + eq
        sel = (rank < float(_TOPK)).astype(jnp.float32)
        sel_ref[...] = sel

        s_ref[...] = jax.lax.dot_general(
            sel, x, (((0,), (0,)), ((), ())),
            preferred_element_type=jnp.float32)
        const_ref[...] = jnp.zeros((1, OUT), jnp.float32)

    s_row = s_ref[pl.ds(e, 1), :]
    iota_be = jax.lax.broadcasted_iota(jnp.int32, (E, 1), 0)
    be_row = jnp.sum(be_ref[...] * (iota_be == e).astype(jnp.float32),
                     axis=0, keepdims=True)
    mean_e = jax.lax.dot(s_row, we_ref[0],
                         preferred_element_type=jnp.float32)
    mean_e = mean_e * jnp.float32(1.0 / B) + be_row
    m = jnp.max(mean_e, axis=1, keepdims=True)
    ex = jnp.exp(mean_e - m)
    g_e = ex / jnp.sum(ex, axis=1, keepdims=True)
    const_ref[...] += be_row * g_e

    iota8 = jax.lax.broadcasted_iota(jnp.int32, (B, E), 1)
    selcol = jnp.sum(sel_ref[...] * (iota8 == e).astype(jnp.float32),
                     axis=1, keepdims=True)
    contrib = jax.lax.dot(x, we_ref[0],
                          preferred_element_type=jnp.float32) * g_e * selcol

    @pl.when(e == 0)
    def _():
        final_ref[...] = contrib

    @pl.when(e != 0)
    def _():
        final_ref[...] = final_ref[...] + contrib

    @pl.when(e == E - 1)
    def _():
        final_ref[...] = final_ref[...] + const_ref[...]


def _make_probs_sc(B, E):
    chunk = B // (_NC * _NS)
    groups = chunk // _L
    mesh = plsc.VectorSubcoreMesh(core_axis_name="c", subcore_axis_name="s")

    @functools.partial(
        pl.kernel, mesh=mesh,
        out_type=jax.ShapeDtypeStruct((_TOPK, B), jnp.float32),
        scratch_types=(
            [pltpu.VMEM((chunk,), jnp.float32) for _ in range(E)]
            + [pltpu.VMEM((chunk,), jnp.float32) for _ in range(_TOPK)]
        ),
    )
    def probs_k(scores_t_hbm, probs_t_hbm, *bufs):
        sbufs = bufs[:E]
        p1buf, p2buf = bufs[E], bufs[E + 1]
        wid = lax.axis_index("s") * _NC + lax.axis_index("c")
        base = wid * chunk
        for e in range(E):
            pltpu.sync_copy(scores_t_hbm.at[e, pl.ds(base, chunk)], sbufs[e])
        one = jnp.full((_L,), 1.0, jnp.float32)
        zero = jnp.full((_L,), 0.0, jnp.float32)
        big = jnp.float32(1e30)
        for j in range(groups):
            sl = pl.ds(j * _L, _L)
            s = [sbufs[e][sl] for e in range(E)]
            m1 = s[0]
            for e in range(1, E):
                m1 = jnp.maximum(m1, s[e])
            found = jnp.where(s[0] == m1, one, zero)
            m2 = s[0] - found * big
            for e in range(1, E):
                eq = jnp.where(s[e] == m1, one, zero)
                is_first = eq * (one - found)
                m2 = jnp.maximum(m2, s[e] - is_first * big)
                found = found + is_first
            p1 = 1.0 / (1.0 + jnp.exp(m2 - m1))
            p1buf[sl] = p1
            p2buf[sl] = 1.0 - p1
        pltpu.sync_copy(p1buf, probs_t_hbm.at[0, pl.ds(base, chunk)])
        pltpu.sync_copy(p2buf, probs_t_hbm.at[1, pl.ds(base, chunk)])

    return probs_k


def kernel(inputs, W_router, b_router, W_experts, b_experts):
    B, D = inputs.shape
    E, _, OUT = W_experts.shape
    br2 = b_router.reshape(1, E)

    final, scores = pl.pallas_call(
        _fused_body,
        grid=(E,),
        in_specs=[
            pl.BlockSpec((B, D), lambda e: (0, 0)),
            pl.BlockSpec((D, E), lambda e: (0, 0)),
            pl.BlockSpec((1, E), lambda e: (0, 0)),
            pl.BlockSpec((1, D, OUT), lambda e: (e, 0, 0)),
            pl.BlockSpec((E, OUT), lambda e: (0, 0)),
        ],
        out_specs=[
            pl.BlockSpec((B, OUT), lambda e: (0, 0)),
            pl.BlockSpec((E, B), lambda e: (0, 0)),
        ],
        out_shape=[
            jax.ShapeDtypeStruct((B, OUT), jnp.float32),
            jax.ShapeDtypeStruct((E, B), jnp.float32),
        ],
        scratch_shapes=[
            pltpu.VMEM((B, E), jnp.float32),
            pltpu.VMEM((E, D), jnp.float32),
            pltpu.VMEM((1, OUT), jnp.float32),
        ],
        compiler_params=pltpu.CompilerParams(
            dimension_semantics=("arbitrary",),
        ),
    )(inputs, W_router, br2, W_experts, b_experts)

    probs_t = _make_probs_sc(B, E)(scores)
    return final, probs_t.T


# speedup vs baseline: 1.0257x; 1.0257x over previous
import functools

import jax
import jax.numpy as jnp
from jax import lax
from jax.experimental import pallas as pl
from jax.experimental.pallas import tpu as pltpu
from jax.experimental.pallas import tpu_sc as plsc

_TOPK = 2
_NC, _NS, _L = 2, 16, 16


def _router_body(x_ref, wr_ref, br_ref, scores_t_ref, sel_ref, s_ref):
    B, D = x_ref.shape
    E = wr_ref.shape[1]
    x = x_ref[...]
    scores = jax.lax.dot(x, wr_ref[...],
                         preferred_element_type=jnp.float32)
    scores = scores + br_ref[...]
    scores_t = jax.lax.dot_general(
        wr_ref[...], x, (((0,), (1,)), ((), ())),
        preferred_element_type=jnp.float32)
    scores_t_ref[...] = scores_t + br_ref[...].reshape(E, 1)

    iota_e = jax.lax.broadcasted_iota(jnp.int32, (B, E), 1)
    rank = jnp.zeros((B, E), jnp.float32)
    for ep in range(8):
        sp = scores[:, ep:ep + 1]
        gt = (sp > scores).astype(jnp.float32)
        eq = jnp.logical_and(sp == scores, ep < iota_e).astype(jnp.float32)
        rank = rank + gt + eq
    sel = (rank < float(_TOPK)).astype(jnp.float32)
    sel_ref[...] = sel

    s_ref[...] = jax.lax.dot_general(
        sel, x, (((0,), (0,)), ((), ())),
        preferred_element_type=jnp.float32)


def _expert_body(x_ref, we_ref, be_ref, sel_ref, s_ref,
                 final_ref, const_ref):
    e = pl.program_id(0)
    E = pl.num_programs(0)
    B, D = x_ref.shape
    OUT = we_ref.shape[2]
    x = x_ref[...]

    @pl.when(e == 0)
    def _():
        const_ref[...] = jnp.zeros((1, OUT), jnp.float32)

    s_row = s_ref[pl.ds(e, 1), :]
    iota_be = jax.lax.broadcasted_iota(jnp.int32, (E, 1), 0)
    be_row = jnp.sum(be_ref[...] * (iota_be == e).astype(jnp.float32),
                     axis=0, keepdims=True)
    mean_e = jax.lax.dot(s_row, we_ref[0],
                         preferred_element_type=jnp.float32)
    mean_e = mean_e * jnp.float32(1.0 / B) + be_row
    m = jnp.max(mean_e, axis=1, keepdims=True)
    ex = jnp.exp(mean_e - m)
    g_e = ex / jnp.sum(ex, axis=1, keepdims=True)
    const_ref[...] += be_row * g_e

    iota8 = jax.lax.broadcasted_iota(jnp.int32, (B, E), 1)
    selcol = jnp.sum(sel_ref[...] * (iota8 == e).astype(jnp.float32),
                     axis=1, keepdims=True)
    contrib = jax.lax.dot(x, we_ref[0],
                          preferred_element_type=jnp.float32) * g_e * selcol

    @pl.when(e == 0)
    def _():
        final_ref[...] = contrib

    @pl.when(e != 0)
    def _():
        final_ref[...] = final_ref[...] + contrib

    @pl.when(e == E - 1)
    def _():
        final_ref[...] = final_ref[...] + const_ref[...]


def _make_probs_sc(B, E):
    chunk = B // (_NC * _NS)
    groups = chunk // _L
    mesh = plsc.VectorSubcoreMesh(core_axis_name="c", subcore_axis_name="s")

    @functools.partial(
        pl.kernel, mesh=mesh,
        out_type=jax.ShapeDtypeStruct((_TOPK, B), jnp.float32),
        scratch_types=(
            [pltpu.VMEM((chunk,), jnp.float32) for _ in range(E)]
            + [pltpu.VMEM((chunk,), jnp.float32) for _ in range(_TOPK)]
        ),
    )
    def probs_k(scores_t_hbm, probs_t_hbm, *bufs):
        sbufs = bufs[:E]
        p1buf, p2buf = bufs[E], bufs[E + 1]
        wid = lax.axis_index("s") * _NC + lax.axis_index("c")
        base = wid * chunk
        for e in range(E):
            pltpu.sync_copy(scores_t_hbm.at[e, pl.ds(base, chunk)], sbufs[e])
        one = jnp.full((_L,), 1.0, jnp.float32)
        zero = jnp.full((_L,), 0.0, jnp.float32)
        big = jnp.float32(1e30)
        for j in range(groups):
            sl = pl.ds(j * _L, _L)
            s = [sbufs[e][sl] for e in range(E)]
            m1 = s[0]
            for e in range(1, E):
                m1 = jnp.maximum(m1, s[e])
            found = jnp.where(s[0] == m1, one, zero)
            m2 = s[0] - found * big
            for e in range(1, E):
                eq = jnp.where(s[e] == m1, one, zero)
                is_first = eq * (one - found)
                m2 = jnp.maximum(m2, s[e] - is_first * big)
                found = found + is_first
            p1 = 1.0 / (1.0 + jnp.exp(m2 - m1))
            p1buf[sl] = p1
            p2buf[sl] = 1.0 - p1
        pltpu.sync_copy(p1buf, probs_t_hbm.at[0, pl.ds(base, chunk)])
        pltpu.sync_copy(p2buf, probs_t_hbm.at[1, pl.ds(base, chunk)])

    return probs_k


def kernel(inputs, W_router, b_router, W_experts, b_experts):
    B, D = inputs.shape
    E, _, OUT = W_experts.shape
    br2 = b_router.reshape(1, E)

    scores_t, sel, s = pl.pallas_call(
        _router_body,
        out_shape=[
            jax.ShapeDtypeStruct((E, B), jnp.float32),
            jax.ShapeDtypeStruct((B, E), jnp.float32),
            jax.ShapeDtypeStruct((E, D), jnp.float32),
        ],
    )(inputs, W_router, br2)

    probs_t = _make_probs_sc(B, E)(scores_t)

    final = pl.pallas_call(
        _expert_body,
        grid=(E,),
        in_specs=[
            pl.BlockSpec((B, D), lambda e: (0, 0)),
            pl.BlockSpec((1, D, OUT), lambda e: (e, 0, 0)),
            pl.BlockSpec((E, OUT), lambda e: (0, 0)),
            pl.BlockSpec((B, E), lambda e: (0, 0)),
            pl.BlockSpec((E, D), lambda e: (0, 0)),
        ],
        out_specs=pl.BlockSpec((B, OUT), lambda e: (0, 0)),
        out_shape=jax.ShapeDtypeStruct((B, OUT), jnp.float32),
        scratch_shapes=[
            pltpu.VMEM((1, OUT), jnp.float32),
        ],
        compiler_params=pltpu.CompilerParams(
            dimension_semantics=("arbitrary",),
        ),
    )(inputs, W_experts, b_experts, sel, s)

    return final, probs_t.T


# speedup vs baseline: 1.3772x; 1.3427x over previous
import jax
import jax.numpy as jnp
from jax.experimental import pallas as pl
from jax.experimental.pallas import tpu as pltpu

_TOPK = 2


def _fused_body(x_ref, wr_ref, br_ref, we_ref, be_ref,
                final_ref, probs_ref,
                sel_ref, s_ref, const_ref):
    e = pl.program_id(0)
    E = pl.num_programs(0)
    B, D = x_ref.shape
    OUT = we_ref.shape[2]
    x = x_ref[...]

    @pl.when(e == 0)
    def _():
        scores = jax.lax.dot(x, wr_ref[...],
                             preferred_element_type=jnp.float32)
        scores = scores + br_ref[...]
        iota_e = jax.lax.broadcasted_iota(jnp.int32, (B, E), 1)
        rank = jnp.zeros((B, E), jnp.float32)
        for ep in range(8):
            sp = scores[:, ep:ep + 1]
            gt = (sp > scores).astype(jnp.float32)
            eq = jnp.logical_and(sp == scores, ep < iota_e).astype(jnp.float32)
            rank = rank + gt + eq
        sel = (rank < float(_TOPK)).astype(jnp.float32)
        sel_ref[...] = sel

        neg = jnp.float32(-1e30)
        v1 = jnp.max(jnp.where(rank == 0.0, scores, neg), axis=1,
                     keepdims=True)
        v2 = jnp.max(jnp.where(rank == 1.0, scores, neg), axis=1,
                     keepdims=True)
        p1 = 1.0 / (1.0 + jnp.exp(v2 - v1))
        probs_ref[...] = jnp.concatenate([p1, 1.0 - p1], axis=1)

        s_ref[...] = jax.lax.dot_general(
            sel, x, (((0,), (0,)), ((), ())),
            preferred_element_type=jnp.float32)
        const_ref[...] = jnp.zeros((1, OUT), jnp.float32)

    s_row = s_ref[pl.ds(e, 1), :]
    iota_be = jax.lax.broadcasted_iota(jnp.int32, (E, 1), 0)
    be_row = jnp.sum(be_ref[...] * (iota_be == e).astype(jnp.float32),
                     axis=0, keepdims=True)
    mean_e = jax.lax.dot(s_row, we_ref[0],
                         preferred_element_type=jnp.float32)
    mean_e = mean_e * jnp.float32(1.0 / B) + be_row
    m = jnp.max(mean_e, axis=1, keepdims=True)
    ex = jnp.exp(mean_e - m)
    g_e = ex / jnp.sum(ex, axis=1, keepdims=True)
    const_ref[...] += be_row * g_e

    iota8 = jax.lax.broadcasted_iota(jnp.int32, (B, E), 1)
    selcol = jnp.sum(sel_ref[...] * (iota8 == e).astype(jnp.float32),
                     axis=1, keepdims=True)
    contrib = jax.lax.dot(x, we_ref[0],
                          preferred_element_type=jnp.float32) * g_e * selcol

    @pl.when(e == 0)
    def _():
        final_ref[...] = contrib

    @pl.when(e != 0)
    def _():
        final_ref[...] = final_ref[...] + contrib

    @pl.when(e == E - 1)
    def _():
        final_ref[...] = final_ref[...] + const_ref[...]


def kernel(inputs, W_router, b_router, W_experts, b_experts):
    B, D = inputs.shape
    E, _, OUT = W_experts.shape
    br2 = b_router.reshape(1, E)

    final, probs = pl.pallas_call(
        _fused_body,
        grid=(E,),
        in_specs=[
            pl.BlockSpec((B, D), lambda e: (0, 0)),
            pl.BlockSpec((D, E), lambda e: (0, 0)),
            pl.BlockSpec((1, E), lambda e: (0, 0)),
            pl.BlockSpec((1, D, OUT), lambda e: (e, 0, 0)),
            pl.BlockSpec((E, OUT), lambda e: (0, 0)),
        ],
        out_specs=[
            pl.BlockSpec((B, OUT), lambda e: (0, 0)),
            pl.BlockSpec((B, _TOPK), lambda e: (0, 0)),
        ],
        out_shape=[
            jax.ShapeDtypeStruct((B, OUT), jnp.float32),
            jax.ShapeDtypeStruct((B, _TOPK), jnp.float32),
        ],
        scratch_shapes=[
            pltpu.VMEM((B, E), jnp.float32),
            pltpu.VMEM((E, D), jnp.float32),
            pltpu.VMEM((1, OUT), jnp.float32),
        ],
        compiler_params=pltpu.CompilerParams(
            dimension_semantics=("arbitrary",),
        ),
    )(inputs, W_router, br2, W_experts, b_experts)

    return final, probs
